# Initial kernel scaffold; baseline (speedup 1.0000x reference)
import jax
import jax.numpy as jnp
from jax.experimental import pallas as pl


def kernel(x, edge_index, hidden_state, cell_state, W_gcn, b_gcn, W_ih, W_hh, b_ih, b_hh, W_fc, b_fc):
    raise NotImplementedError("write your pallas kernel here")


# speedup vs baseline: 8.1073x; 8.1073x over previous
import functools

import jax
import jax.numpy as jnp
from jax import lax
from jax.experimental import pallas as pl
from jax.experimental.pallas import tpu as pltpu
from jax.experimental.pallas import tpu_sc as plsc

N = 10000
E = 160000
D_IN = 128
D_H = 64
D_OUT = 3

NPAD = 10240
ROWS_PER_TILE = 640
CHUNK = 128
E_PER_SC = E // 2
CHUNKS_PER_SC = E_PER_SC // CHUNK
GMAX = 40

D_PAD = 128


def _deg_body(dst_hbm, ones_hbm, zeros_hbm, out_hbm, dst_v, ones_v, deg_sh):
    c = lax.axis_index("c")
    s = lax.axis_index("s")
    pltpu.sync_copy(zeros_hbm.at[pl.ds(s * ROWS_PER_TILE, ROWS_PER_TILE)],
                    deg_sh.at[pl.ds(s * ROWS_PER_TILE, ROWS_PER_TILE)])
    pltpu.sync_copy(ones_hbm, ones_v)
    plsc.subcore_barrier()

    def body(g, carry):
        k = s + 16 * g

        @pl.when(k < CHUNKS_PER_SC)
        def _():
            base = c * E_PER_SC + k * CHUNK
            pltpu.sync_copy(dst_hbm.at[pl.ds(base, CHUNK)], dst_v)
            pltpu.sync_copy(ones_v, deg_sh.at[dst_v], add=True)

        return carry

    lax.fori_loop(0, GMAX, body, 0)
    plsc.subcore_barrier()
    pltpu.sync_copy(deg_sh.at[pl.ds(s * ROWS_PER_TILE, ROWS_PER_TILE)],
                    out_hbm.at[c, pl.ds(s * ROWS_PER_TILE, ROWS_PER_TILE)])


@functools.cache
def _deg_call():
    return pl.kernel(
        _deg_body,
        mesh=plsc.VectorSubcoreMesh(core_axis_name="c", subcore_axis_name="s"),
        out_type=jax.ShapeDtypeStruct((2, NPAD, D_PAD), jnp.float32),
        scratch_types=[
            pltpu.VMEM((CHUNK,), jnp.int32),
            pltpu.VMEM((CHUNK, D_PAD), jnp.float32),
            pltpu.VMEM_SHARED((NPAD, D_PAD), jnp.float32),
        ],
    )


def _agg_body(src_hbm, dst_hbm, hp_hbm, zeros_hbm, out_hbm,
              src_v, dst_v, rows_v, acc_sh):
    c = lax.axis_index("c")
    s = lax.axis_index("s")
    pltpu.sync_copy(zeros_hbm.at[pl.ds(s * ROWS_PER_TILE, ROWS_PER_TILE)],
                    acc_sh.at[pl.ds(s * ROWS_PER_TILE, ROWS_PER_TILE)])
    plsc.subcore_barrier()

    def body(g, carry):
        k = s + 16 * g

        @pl.when(k < CHUNKS_PER_SC)
        def _():
            base = c * E_PER_SC + k * CHUNK
            pltpu.sync_copy(src_hbm.at[pl.ds(base, CHUNK)], src_v)
            pltpu.sync_copy(dst_hbm.at[pl.ds(base, CHUNK)], dst_v)
            pltpu.sync_copy(hp_hbm.at[src_v], rows_v)
            pltpu.sync_copy(rows_v, acc_sh.at[dst_v], add=True)

        return carry

    lax.fori_loop(0, GMAX, body, 0)
    plsc.subcore_barrier()
    pltpu.sync_copy(acc_sh.at[pl.ds(s * ROWS_PER_TILE, ROWS_PER_TILE)],
                    out_hbm.at[c, pl.ds(s * ROWS_PER_TILE, ROWS_PER_TILE)])


@functools.cache
def _agg_call():
    return pl.kernel(
        _agg_body,
        mesh=plsc.VectorSubcoreMesh(core_axis_name="c", subcore_axis_name="s"),
        out_type=jax.ShapeDtypeStruct((2, NPAD, D_PAD), jnp.float32),
        scratch_types=[
            pltpu.VMEM((CHUNK,), jnp.int32),
            pltpu.VMEM((CHUNK,), jnp.int32),
            pltpu.VMEM((CHUNK, D_PAD), jnp.float32),
            pltpu.VMEM_SHARED((NPAD, D_PAD), jnp.float32),
        ],
    )


_BLK = 2000


def _prep_body(x_ref, w_ref, deg_ref, hp_ref, dv_ref):
    h = jnp.dot(x_ref[...], w_ref[...], preferred_element_type=jnp.float32)
    deg = deg_ref[0, :, 0] + deg_ref[1, :, 0] + 1.0
    dinv = lax.rsqrt(deg)[:, None]
    hp = h * dinv
    hp_ref[...] = jnp.concatenate(
        [hp, jnp.zeros((_BLK, D_PAD - D_H), jnp.float32)], axis=1)
    dv_ref[...] = jnp.broadcast_to(dinv, (_BLK, D_H))


def _prep_call(x, w_gcn, deg2):
    return pl.pallas_call(
        _prep_body,
        grid=(N // _BLK,),
        in_specs=[
            pl.BlockSpec((_BLK, D_IN), lambda i: (i, 0)),
            pl.BlockSpec((D_IN, D_H), lambda i: (0, 0)),
            pl.BlockSpec((2, _BLK, D_PAD), lambda i: (0, i, 0)),
        ],
        out_specs=[
            pl.BlockSpec((_BLK, D_PAD), lambda i: (i, 0)),
            pl.BlockSpec((_BLK, D_H), lambda i: (i, 0)),
        ],
        out_shape=[
            jax.ShapeDtypeStruct((N, D_PAD), jnp.float32),
            jax.ShapeDtypeStruct((N, D_H), jnp.float32),
        ],
    )(x, w_gcn, deg2)


def _final_body(acc_ref, hp_ref, dv_ref, bg_ref, wih_ref, whh_ref, bihh_ref,
                wfc_ref, bfc_ref, h0_ref, c0_ref,
                out_ref, hn_ref, cn_ref, gx_ref, ys_ref):
    g = dv_ref[...] * (acc_ref[0] + acc_ref[1] + hp_ref[...]) + bg_ref[...]
    gx_ref[...] = lax.dot_general(
        g, wih_ref[...], (((1,), (1,)), ((), ())),
        preferred_element_type=jnp.float32) + bihh_ref[...]
    whh = whh_ref[...]

    def step(t, carry):
        h, c = carry
        gates = gx_ref[pl.ds(t, 1), :] + lax.dot_general(
            h, whh, (((1,), (1,)), ((), ())),
            preferred_element_type=jnp.float32)
        i = jax.nn.sigmoid(gates[:, 0:D_H])
        f = jax.nn.sigmoid(gates[:, D_H:2 * D_H])
        gg = jnp.tanh(gates[:, 2 * D_H:3 * D_H])
        o = jax.nn.sigmoid(gates[:, 3 * D_H:4 * D_H])
        c = f * c + i * gg
        h = o * jnp.tanh(c)
        ys_ref[pl.ds(t, 1), :] = h
        return (h, c)

    h, c = lax.fori_loop(0, N, step, (h0_ref[...], c0_ref[...]))
    out_ref[...] = lax.dot_general(
        ys_ref[...], wfc_ref[...], (((1,), (1,)), ((), ())),
        preferred_element_type=jnp.float32) + bfc_ref[...]
    hn_ref[...] = h
    cn_ref[...] = c


def _final_call(acc2, hp, dv, b_gcn, w_ih, w_hh, bihh, w_fc, b_fc, h0, c0):
    return pl.pallas_call(
        _final_body,
        out_shape=[
            jax.ShapeDtypeStruct((N, D_OUT), jnp.float32),
            jax.ShapeDtypeStruct((1, D_H), jnp.float32),
            jax.ShapeDtypeStruct((1, D_H), jnp.float32),
        ],
        scratch_shapes=[
            pltpu.VMEM((N, 4 * D_H), jnp.float32),
            pltpu.VMEM((N, D_H), jnp.float32),
        ],
    )(acc2, hp, dv, b_gcn, w_ih, w_hh, bihh, w_fc, b_fc, h0, c0)


def kernel(x, edge_index, hidden_state, cell_state, W_gcn, b_gcn,
           W_ih, W_hh, b_ih, b_hh, W_fc, b_fc):
    src = edge_index[0].astype(jnp.int32)
    dst = edge_index[1].astype(jnp.int32)

    ones128 = jnp.ones((CHUNK, D_PAD), jnp.float32)
    zeros128 = jnp.zeros((NPAD, D_PAD), jnp.float32)

    deg2 = _deg_call()(dst, ones128, zeros128)
    hp, dv = _prep_call(x, W_gcn, deg2[:, :N, :])
    acc2 = _agg_call()(src, dst, hp, zeros128)

    bihh = (b_ih + b_hh).reshape(1, 4 * D_H)
    out, hn, cn = _final_call(
        acc2[:, :N, :D_H], hp[:, :D_H], dv, b_gcn.reshape(1, D_H), W_ih, W_hh, bihh,
        W_fc, b_fc.reshape(1, D_OUT),
        hidden_state.reshape(1, D_H), cell_state.reshape(1, D_H))
    return out, hn.reshape(1, 1, D_H), cn.reshape(1, 1, D_H)


# speedup vs baseline: 16.4603x; 2.0303x over previous
import functools

import jax
import jax.numpy as jnp
from jax import lax
from jax.experimental import pallas as pl
from jax.experimental.pallas import tpu as pltpu
from jax.experimental.pallas import tpu_sc as plsc

N = 10000
E = 160000
D_IN = 128
D_H = 64
D_OUT = 3

NPAD = 10240
ROWS_PER_TILE = 640
CHUNK = 128
E_PER_SC = E // 2
CHUNKS_PER_SC = E_PER_SC // CHUNK
GMAX = 40

D_PAD = 128


def _deg_body(dst_hbm, ones_hbm, zeros_hbm, out_hbm, dst_v, ones_v, deg_sh):
    c = lax.axis_index("c")
    s = lax.axis_index("s")
    pltpu.sync_copy(zeros_hbm.at[pl.ds(s * ROWS_PER_TILE, ROWS_PER_TILE)],
                    deg_sh.at[pl.ds(s * ROWS_PER_TILE, ROWS_PER_TILE)])
    pltpu.sync_copy(ones_hbm, ones_v)
    plsc.subcore_barrier()

    def body(g, carry):
        k = s + 16 * g

        @pl.when(k < CHUNKS_PER_SC)
        def _():
            base = c * E_PER_SC + k * CHUNK
            pltpu.sync_copy(dst_hbm.at[pl.ds(base, CHUNK)], dst_v)
            pltpu.sync_copy(ones_v, deg_sh.at[dst_v], add=True)

        return carry

    lax.fori_loop(0, GMAX, body, 0)
    plsc.subcore_barrier()
    pltpu.sync_copy(deg_sh.at[pl.ds(s * ROWS_PER_TILE, ROWS_PER_TILE)],
                    out_hbm.at[c, pl.ds(s * ROWS_PER_TILE, ROWS_PER_TILE)])


@functools.cache
def _deg_call():
    return pl.kernel(
        _deg_body,
        mesh=plsc.VectorSubcoreMesh(core_axis_name="c", subcore_axis_name="s"),
        out_type=jax.ShapeDtypeStruct((2, NPAD, D_PAD), jnp.float32),
        scratch_types=[
            pltpu.VMEM((CHUNK,), jnp.int32),
            pltpu.VMEM((CHUNK, D_PAD), jnp.float32),
            pltpu.VMEM_SHARED((NPAD, D_PAD), jnp.float32),
        ],
    )


def _agg_body(src_hbm, dst_hbm, hp_hbm, zeros_hbm, out_hbm,
              src_v, dst_v, rows_v, acc_sh):
    c = lax.axis_index("c")
    s = lax.axis_index("s")
    pltpu.sync_copy(zeros_hbm.at[pl.ds(s * ROWS_PER_TILE, ROWS_PER_TILE)],
                    acc_sh.at[pl.ds(s * ROWS_PER_TILE, ROWS_PER_TILE)])
    plsc.subcore_barrier()

    def body(g, carry):
        k = s + 16 * g

        @pl.when(k < CHUNKS_PER_SC)
        def _():
            base = c * E_PER_SC + k * CHUNK
            pltpu.sync_copy(src_hbm.at[pl.ds(base, CHUNK)], src_v)
            pltpu.sync_copy(dst_hbm.at[pl.ds(base, CHUNK)], dst_v)
            pltpu.sync_copy(hp_hbm.at[src_v], rows_v)
            pltpu.sync_copy(rows_v, acc_sh.at[dst_v], add=True)

        return carry

    lax.fori_loop(0, GMAX, body, 0)
    plsc.subcore_barrier()
    pltpu.sync_copy(acc_sh.at[pl.ds(s * ROWS_PER_TILE, ROWS_PER_TILE)],
                    out_hbm.at[c, pl.ds(s * ROWS_PER_TILE, ROWS_PER_TILE)])


@functools.cache
def _agg_call():
    return pl.kernel(
        _agg_body,
        mesh=plsc.VectorSubcoreMesh(core_axis_name="c", subcore_axis_name="s"),
        out_type=jax.ShapeDtypeStruct((2, NPAD, D_PAD), jnp.float32),
        scratch_types=[
            pltpu.VMEM((CHUNK,), jnp.int32),
            pltpu.VMEM((CHUNK,), jnp.int32),
            pltpu.VMEM((CHUNK, D_PAD), jnp.float32),
            pltpu.VMEM_SHARED((NPAD, D_PAD), jnp.float32),
        ],
    )


_BLK = 2000


def _prep_body(x_ref, w_ref, deg_ref, hp_ref, dv_ref):
    h = jnp.dot(x_ref[...], w_ref[...], preferred_element_type=jnp.float32)
    deg = deg_ref[0, :, 0] + deg_ref[1, :, 0] + 1.0
    dinv = lax.rsqrt(deg)[:, None]
    hp = h * dinv
    hp_ref[...] = jnp.concatenate(
        [hp, jnp.zeros((_BLK, D_PAD - D_H), jnp.float32)], axis=1)
    dv_ref[...] = jnp.broadcast_to(dinv, (_BLK, D_H))


def _prep_call(x, w_gcn, deg2):
    return pl.pallas_call(
        _prep_body,
        grid=(N // _BLK,),
        in_specs=[
            pl.BlockSpec((_BLK, D_IN), lambda i: (i, 0)),
            pl.BlockSpec((D_IN, D_H), lambda i: (0, 0)),
            pl.BlockSpec((2, _BLK, D_PAD), lambda i: (0, i, 0)),
        ],
        out_specs=[
            pl.BlockSpec((_BLK, D_PAD), lambda i: (i, 0)),
            pl.BlockSpec((_BLK, D_H), lambda i: (i, 0)),
        ],
        out_shape=[
            jax.ShapeDtypeStruct((N, D_PAD), jnp.float32),
            jax.ShapeDtypeStruct((N, D_H), jnp.float32),
        ],
    )(x, w_gcn, deg2)


def _gx_body(acc_ref, hp_ref, dv_ref, bg_ref, wih4_ref, bihh4_ref,
             gxi_ref, gxf_ref, gxg_ref, gxo_ref):
    g = dv_ref[...] * (acc_ref[0] + acc_ref[1] + hp_ref[...]) + bg_ref[...]
    for s, r in enumerate([gxi_ref, gxf_ref, gxg_ref, gxo_ref]):
        r[...] = lax.dot_general(
            g, wih4_ref[s], (((1,), (0,)), ((), ())),
            preferred_element_type=jnp.float32) + bihh4_ref[s:s + 1, :]


def _gx_call(acc2, hp, dv, b_gcn, wih4, bihh4):
    spec = pl.BlockSpec((_BLK, D_H), lambda i: (i, 0))
    return pl.pallas_call(
        _gx_body,
        grid=(N // _BLK,),
        in_specs=[
            pl.BlockSpec((2, _BLK, D_H), lambda i: (0, i, 0)),
            spec,
            spec,
            pl.BlockSpec((1, D_H), lambda i: (0, 0)),
            pl.BlockSpec((4, D_H, D_H), lambda i: (0, 0, 0)),
            pl.BlockSpec((4, D_H), lambda i: (0, 0)),
        ],
        out_specs=[spec, spec, spec, spec],
        out_shape=[jax.ShapeDtypeStruct((N, D_H), jnp.float32)
                   for _ in range(4)],
    )(acc2, hp, dv, b_gcn, wih4, bihh4)


_TB = 200
_NB = N // _TB


def _final_body(gxi_ref, gxf_ref, gxg_ref, gxo_ref, whh4_ref,
                wfc_ref, bfc_ref, h0_ref, c0_ref,
                out_ref, hn_ref, cn_ref, ys_ref, h_ref, c_ref):
    b = pl.program_id(0)

    @pl.when(b == 0)
    def _():
        h_ref[...] = h0_ref[...]
        c_ref[...] = c0_ref[...]

    wti = whh4_ref[0]
    wtf = whh4_ref[1]
    wtg = whh4_ref[2]
    wto = whh4_ref[3]

    def mv(h, w):
        return lax.dot_general(h, w, (((1,), (0,)), ((), ())),
                               preferred_element_type=jnp.float32)

    def step(t, carry):
        h, c = carry
        t1 = pl.ds(t, 1)
        i = 0.5 + 0.5 * jnp.tanh(gxi_ref[t1, :] + mv(h, wti))
        f = 0.5 + 0.5 * jnp.tanh(gxf_ref[t1, :] + mv(h, wtf))
        gg = jnp.tanh(gxg_ref[t1, :] + mv(h, wtg))
        o = 0.5 + 0.5 * jnp.tanh(gxo_ref[t1, :] + mv(h, wto))
        c = f * c + i * gg
        h = o * jnp.tanh(c)
        ys_ref[t1, :] = h
        return (h, c)

    h, c = lax.fori_loop(0, _TB, step, (h_ref[...], c_ref[...]), unroll=8)
    h_ref[...] = h
    c_ref[...] = c
    out_ref[...] = lax.dot_general(
        ys_ref[...], wfc_ref[...], (((1,), (1,)), ((), ())),
        preferred_element_type=jnp.float32) + bfc_ref[...]
    hn_ref[...] = h
    cn_ref[...] = c


def _final_call(gxi, gxf, gxg, gxo, whh4, w_fc, b_fc, h0, c0):
    tspec = pl.BlockSpec((_TB, D_H), lambda b: (b, 0))
    return pl.pallas_call(
        _final_body,
        grid=(_NB,),
        in_specs=[
            tspec, tspec, tspec, tspec,
            pl.BlockSpec((4, D_H, D_H), lambda b: (0, 0, 0)),
            pl.BlockSpec((D_OUT, D_H), lambda b: (0, 0)),
            pl.BlockSpec((1, D_OUT), lambda b: (0, 0)),
            pl.BlockSpec((1, D_H), lambda b: (0, 0)),
            pl.BlockSpec((1, D_H), lambda b: (0, 0)),
        ],
        out_specs=[
            pl.BlockSpec((_TB, D_OUT), lambda b: (b, 0)),
            pl.BlockSpec((1, D_H), lambda b: (0, 0)),
            pl.BlockSpec((1, D_H), lambda b: (0, 0)),
        ],
        out_shape=[
            jax.ShapeDtypeStruct((N, D_OUT), jnp.float32),
            jax.ShapeDtypeStruct((1, D_H), jnp.float32),
            jax.ShapeDtypeStruct((1, D_H), jnp.float32),
        ],
        scratch_shapes=[
            pltpu.VMEM((_TB, D_H), jnp.float32),
            pltpu.VMEM((1, D_H), jnp.float32),
            pltpu.VMEM((1, D_H), jnp.float32),
        ],
    )(gxi, gxf, gxg, gxo, whh4, w_fc, b_fc, h0, c0)


def kernel(x, edge_index, hidden_state, cell_state, W_gcn, b_gcn,
           W_ih, W_hh, b_ih, b_hh, W_fc, b_fc):
    src = edge_index[0].astype(jnp.int32)
    dst = edge_index[1].astype(jnp.int32)

    ones128 = jnp.ones((CHUNK, D_PAD), jnp.float32)
    zeros128 = jnp.zeros((NPAD, D_PAD), jnp.float32)

    deg2 = _deg_call()(dst, ones128, zeros128)
    hp, dv = _prep_call(x, W_gcn, deg2[:, :N, :])
    acc2 = _agg_call()(src, dst, hp, zeros128)

    gscale = jnp.array([0.5, 0.5, 1.0, 0.5], jnp.float32)
    wih4 = W_ih.reshape(4, D_H, D_H).transpose(0, 2, 1) * gscale[:, None, None]
    whh4 = W_hh.reshape(4, D_H, D_H).transpose(0, 2, 1) * gscale[:, None, None]
    bihh4 = (b_ih + b_hh).reshape(4, D_H) * gscale[:, None]
    gxi, gxf, gxg, gxo = _gx_call(
        acc2[:, :N, :D_H], hp[:, :D_H], dv, b_gcn.reshape(1, D_H),
        wih4, bihh4)
    out, hn, cn = _final_call(
        gxi, gxf, gxg, gxo, whh4,
        W_fc, b_fc.reshape(1, D_OUT),
        hidden_state.reshape(1, D_H), cell_state.reshape(1, D_H))
    return out, hn.reshape(1, 1, D_H), cn.reshape(1, 1, D_H)
